# Initial kernel scaffold; baseline (speedup 1.0000x reference)
import jax
import jax.numpy as jnp
from jax.experimental import pallas as pl


def kernel(token_feat, centers_coarse, centers_fine, g_head, g_ch, g_block, patch_coords):
    raise NotImplementedError("write your pallas kernel here")


# speedup vs baseline: 1.5788x; 1.5788x over previous
import jax
import jax.numpy as jnp
import numpy as np
from jax.experimental import pallas as pl
from jax.experimental.pallas import tpu as pltpu

H_P, W_P = 14, 14
N_TOK = H_P * W_P
EMBED_DIM = 768
NUM_HEADS = 12
DEPTH = 12
HIDDEN_DIM = 3072
R_C, R_F = 4, 8
TAU = 1.0
EPS = 1e-6
ALPHA, BETA, GAMMA = 1.0, 0.5, 0.5
RHO = 0.5
TOK_TEMP = 0.1
B, T = 8, 16
K_TOP = max(1, int(RHO * T * N_TOK))


def _interp_matrix(t_in, t_out):
    src = (np.arange(t_out, dtype=np.float64) + 0.5) * (t_in / float(t_out)) - 0.5
    src = np.clip(src, 0.0, t_in - 1.0)
    lo = np.floor(src).astype(np.int64)
    hi = np.minimum(lo + 1, t_in - 1)
    w = (src - lo).astype(np.float32)
    m = np.zeros((t_out, t_in), dtype=np.float32)
    m[np.arange(t_out), lo] += 1.0 - w
    m[np.arange(t_out), hi] += w
    return m


M2_NP = _interp_matrix(T - 1, T)
M4_NP = _interp_matrix(T - 3, T)


def _region_one_hot_t(coords_t, centers):
    r = centers.shape[0]
    dx = centers[:, 0:1] - coords_t[0:1, :]
    dy = centers[:, 1:2] - coords_t[1:2, :]
    d = jnp.sqrt(jnp.maximum(dx * dx + dy * dy, 0.0))
    rid = jnp.argmin(d, axis=0)
    iot = jax.lax.broadcasted_iota(jnp.int32, (r, d.shape[1]), 0)
    return (iot == rid[None, :].astype(jnp.int32)).astype(jnp.float32)


def _entropy_kernel(x_ref, coords_t_ref, cc_ref, cf_ref,
                    ent1_ref, ent2_ref, ent4_ref, hc_ref, hf_ref,
                    hist_ref):
    t = pl.program_id(1)
    x = x_ref[0, 0]
    m = jnp.max(x, axis=1, keepdims=True)
    e = jnp.exp((x - m) * (1.0 / TAU))
    z = jnp.sum(e, axis=1, keepdims=True)
    p = e / z

    ent1 = -jnp.sum(p * jnp.log(p + EPS), axis=1)
    ent1_ref[0, t, :] = ent1

    r0 = jax.lax.rem(t, 4)
    hist_ref[r0] = p

    @pl.when(t >= 1)
    def _l2():
        p1 = hist_ref[jax.lax.rem(t + 3, 4)]
        q = 0.5 * (p + p1)
        ent2_ref[0, t, :] = -jnp.sum(q * jnp.log(q + EPS), axis=1)

    @pl.when(t == 0)
    def _l2z():
        ent2_ref[0, 0, :] = jnp.zeros((N_TOK,), jnp.float32)

    @pl.when(t >= 3)
    def _l4():
        p1 = hist_ref[jax.lax.rem(t + 3, 4)]
        p2 = hist_ref[jax.lax.rem(t + 2, 4)]
        p3 = hist_ref[jax.lax.rem(t + 1, 4)]
        q = 0.25 * (p + p1 + p2 + p3)
        ent4_ref[0, t, :] = -jnp.sum(q * jnp.log(q + EPS), axis=1)

    @pl.when(t < 3)
    def _l4z():
        ent4_ref[0, t, :] = jnp.zeros((N_TOK,), jnp.float32)

    coords_t = coords_t_ref[...]
    oh_c = _region_one_hot_t(coords_t, cc_ref[...])
    oh_f = _region_one_hot_t(coords_t, cf_ref[...])
    oh = jnp.concatenate([oh_c, oh_f], axis=0)
    cnt = jnp.sum(oh, axis=1, keepdims=True)
    p_sum = jnp.dot(oh, p, preferred_element_type=jnp.float32, precision=jax.lax.Precision.HIGHEST)
    p_reg = p_sum / (cnt + EPS)
    ent_r = -jnp.sum(p_reg * jnp.log(p_reg + EPS), axis=1)
    hc_ref[0, t, :] = ent_r[:R_C]
    hf_ref[0, t, :] = ent_r[R_C:]


def _entropy_pass(x, coords_t, cc, cf):
    n, c = N_TOK, EMBED_DIM
    return pl.pallas_call(
        _entropy_kernel,
        grid=(B, T),
        in_specs=[
            pl.BlockSpec((1, 1, n, c), lambda b, t: (b, t, 0, 0)),
            pl.BlockSpec((2, n), lambda b, t: (0, 0)),
            pl.BlockSpec((R_C, 2), lambda b, t: (0, 0)),
            pl.BlockSpec((R_F, 2), lambda b, t: (0, 0)),
        ],
        out_specs=[
            pl.BlockSpec((1, T, n), lambda b, t: (b, 0, 0)),
            pl.BlockSpec((1, T, n), lambda b, t: (b, 0, 0)),
            pl.BlockSpec((1, T, n), lambda b, t: (b, 0, 0)),
            pl.BlockSpec((1, T, R_C), lambda b, t: (b, 0, 0)),
            pl.BlockSpec((1, T, R_F), lambda b, t: (b, 0, 0)),
        ],
        out_shape=[
            jax.ShapeDtypeStruct((B, T, n), jnp.float32),
            jax.ShapeDtypeStruct((B, T, n), jnp.float32),
            jax.ShapeDtypeStruct((B, T, n), jnp.float32),
            jax.ShapeDtypeStruct((B, T, R_C), jnp.float32),
            jax.ShapeDtypeStruct((B, T, R_F), jnp.float32),
        ],
        scratch_shapes=[pltpu.VMEM((4, n, c), jnp.float32)],
    )(x, coords_t, cc, cf)


def _normalize(h):
    mn = jnp.min(h)
    mx = jnp.max(h)
    return (h - mn) / (mx - mn + EPS)


def _kth_largest(score, k):
    hi0 = jnp.max(score) + 1.0
    lo0 = jnp.zeros((), jnp.float32)

    def body(_, carry):
        lo, hi = carry
        mid = 0.5 * (lo + hi)
        cnt = jnp.sum((score >= mid).astype(jnp.float32))
        ge = cnt >= float(k)
        return jnp.where(ge, mid, lo), jnp.where(ge, hi, mid)

    lo, _ = jax.lax.fori_loop(0, 50, body, (lo0, hi0))
    return lo


def _mask_kernel(ent1_ref, ent2_ref, ent4_ref, hc_ref, hf_ref,
                 coords_t_ref, cc_ref, cf_ref,
                 ghead_ref, gch_ref, gblock_ref, m2_ref, m4_ref,
                 mask_ref, headw_ref, chw_ref, blockw_ref, st_ref, last_ref):
    m2 = m2_ref[...]
    m4 = m4_ref[...]
    coords_t = coords_t_ref[...]
    oh_c = _region_one_hot_t(coords_t, cc_ref[...])
    oh_f = _region_one_hot_t(coords_t, cf_ref[...])

    total = jnp.zeros((), jnp.float32)
    for b in range(B):
        e1 = ent1_ref[b]
        e2 = ent2_ref[b][1:T, :]
        e4 = ent4_ref[b][3:T, :]
        i2 = jnp.dot(m2, e2, preferred_element_type=jnp.float32, precision=jax.lax.Precision.HIGHEST)
        i4 = jnp.dot(m4, e4, preferred_element_type=jnp.float32, precision=jax.lax.Precision.HIGHEST)
        ht = (e1 + i2 + i4) * (1.0 / 3.0)
        ht_n = _normalize(ht)
        hc_n = _normalize(hc_ref[b])
        hf_n = _normalize(hf_ref[b])
        hc_tok = jnp.dot(hc_n, oh_c, preferred_element_type=jnp.float32, precision=jax.lax.Precision.HIGHEST)
        hf_tok = jnp.dot(hf_n, oh_f, preferred_element_type=jnp.float32, precision=jax.lax.Precision.HIGHEST)
        score = ALPHA * ht_n + BETA * hc_tok + GAMMA * hf_tok
        kth = _kth_largest(score, K_TOP)
        mask = jax.nn.sigmoid((score - kth) * (1.0 / TOK_TEMP))
        mask_ref[b] = mask
        total = total + jnp.sum(mask)

    sparsity_token = 1.0 - total / float(B * T * N_TOK)
    head_w = jax.nn.sigmoid(ghead_ref[...])
    ch_w = jax.nn.sigmoid(gch_ref[...])
    block_w = jax.nn.sigmoid(gblock_ref[...])
    headw_ref[...] = head_w
    chw_ref[...] = ch_w
    blockw_ref[...] = block_w
    l_ast = (sparsity_token + (1.0 - jnp.mean(head_w))
             + (1.0 - jnp.mean(ch_w)) + (1.0 - jnp.mean(block_w)))
    st_ref[...] = jnp.reshape(sparsity_token, (1, 1))
    last_ref[...] = jnp.reshape(l_ast, (1, 1))


def _mask_pass(ent1, ent2, ent4, hc, hf, coords_t, cc, cf,
               g_head, g_ch, g_block2d):
    return pl.pallas_call(
        _mask_kernel,
        out_shape=[
            jax.ShapeDtypeStruct((B, T, N_TOK), jnp.float32),
            jax.ShapeDtypeStruct((DEPTH, NUM_HEADS), jnp.float32),
            jax.ShapeDtypeStruct((DEPTH, HIDDEN_DIM), jnp.float32),
            jax.ShapeDtypeStruct((1, DEPTH), jnp.float32),
            jax.ShapeDtypeStruct((1, 1), jnp.float32),
            jax.ShapeDtypeStruct((1, 1), jnp.float32),
        ],
    )(ent1, ent2, ent4, hc, hf, coords_t, cc, cf, g_head, g_ch, g_block2d,
      jnp.asarray(M2_NP), jnp.asarray(M4_NP))


def kernel(token_feat, centers_coarse, centers_fine, g_head, g_ch, g_block,
           patch_coords):
    coords_t = patch_coords.T
    ent1, ent2, ent4, hc, hf = _entropy_pass(
        token_feat, coords_t, centers_coarse, centers_fine)
    mask, head_w, ch_w, block_w2, st, last = _mask_pass(
        ent1, ent2, ent4, hc, hf, coords_t, centers_coarse, centers_fine,
        g_head, g_ch, g_block.reshape(1, DEPTH))
    return (mask, head_w, ch_w, block_w2.reshape(DEPTH),
            st.reshape(()), last.reshape(()))


# speedup vs baseline: 1.7398x; 1.1020x over previous
import jax
import jax.numpy as jnp
import numpy as np
from jax.experimental import pallas as pl
from jax.experimental.pallas import tpu as pltpu

H_P, W_P = 14, 14
N_TOK = H_P * W_P
EMBED_DIM = 768
NUM_HEADS = 12
DEPTH = 12
HIDDEN_DIM = 3072
R_C, R_F = 4, 8
TAU = 1.0
EPS = 1e-6
ALPHA, BETA, GAMMA = 1.0, 0.5, 0.5
RHO = 0.5
TOK_TEMP = 0.1
B, T = 8, 16
K_TOP = max(1, int(RHO * T * N_TOK))


def _interp_matrix(t_in, t_out):
    src = (np.arange(t_out, dtype=np.float64) + 0.5) * (t_in / float(t_out)) - 0.5
    src = np.clip(src, 0.0, t_in - 1.0)
    lo = np.floor(src).astype(np.int64)
    hi = np.minimum(lo + 1, t_in - 1)
    w = (src - lo).astype(np.float32)
    m = np.zeros((t_out, t_in), dtype=np.float32)
    m[np.arange(t_out), lo] += 1.0 - w
    m[np.arange(t_out), hi] += w
    return m


M2_NP = _interp_matrix(T - 1, T)
M4_NP = _interp_matrix(T - 3, T)


def _region_one_hot(coords, centers):
    d = jnp.sqrt(jnp.maximum(
        ((coords[:, None, :] - centers[None, :, :]) ** 2).sum(-1), 0.0))
    rid = jnp.argmin(d, axis=1)
    return (rid[None, :] == jnp.arange(centers.shape[0])[:, None]).astype(
        jnp.float32)


def _entropy_kernel(x_ref, oh_ref,
                    ent1_ref, ent2_ref, ent4_ref, hc_ref, hf_ref,
                    hist_ref):
    t = pl.program_id(1)
    x = x_ref[0, 0]
    m = jnp.max(x, axis=1, keepdims=True)
    e = jnp.exp((x - m) * (1.0 / TAU))
    z = jnp.sum(e, axis=1, keepdims=True)
    p = e / z

    s_prev1 = jnp.where(t >= 1, hist_ref[jax.lax.rem(t + 3, 4)], 0.0)
    s_t = s_prev1 + p

    q1 = s_t - s_prev1
    ent1_ref[0, t, :] = -jnp.sum(q1 * jnp.log(q1 + EPS), axis=1)

    @pl.when(t >= 1)
    def _l2():
        s2 = jnp.where(t >= 2, hist_ref[jax.lax.rem(t + 2, 4)], 0.0)
        q = (s_t - s2) * 0.5
        ent2_ref[0, t, :] = -jnp.sum(q * jnp.log(q + EPS), axis=1)

    @pl.when(t == 0)
    def _l2z():
        ent2_ref[0, 0, :] = jnp.zeros((N_TOK,), jnp.float32)

    @pl.when(t >= 3)
    def _l4():
        s4 = jnp.where(t >= 4, hist_ref[jax.lax.rem(t, 4)], 0.0)
        q = (s_t - s4) * 0.25
        ent4_ref[0, t, :] = -jnp.sum(q * jnp.log(q + EPS), axis=1)

    @pl.when(t < 3)
    def _l4z():
        ent4_ref[0, t, :] = jnp.zeros((N_TOK,), jnp.float32)

    hist_ref[jax.lax.rem(t, 4)] = s_t

    oh = oh_ref[...]
    cnt = jnp.sum(oh, axis=1, keepdims=True)
    p_sum = jnp.dot(oh, p, preferred_element_type=jnp.float32)
    p_reg = p_sum / (cnt + EPS)
    ent_r = -jnp.sum(p_reg * jnp.log(p_reg + EPS), axis=1)
    hc_ref[0, t, :] = ent_r[:R_C]
    hf_ref[0, t, :] = ent_r[R_C:]


def _entropy_pass(x, oh):
    n, c = N_TOK, EMBED_DIM
    return pl.pallas_call(
        _entropy_kernel,
        grid=(B, T),
        in_specs=[
            pl.BlockSpec((1, 1, n, c), lambda b, t: (b, t, 0, 0)),
            pl.BlockSpec((R_C + R_F, n), lambda b, t: (0, 0)),
        ],
        out_specs=[
            pl.BlockSpec((1, T, n), lambda b, t: (b, 0, 0)),
            pl.BlockSpec((1, T, n), lambda b, t: (b, 0, 0)),
            pl.BlockSpec((1, T, n), lambda b, t: (b, 0, 0)),
            pl.BlockSpec((1, T, R_C), lambda b, t: (b, 0, 0)),
            pl.BlockSpec((1, T, R_F), lambda b, t: (b, 0, 0)),
        ],
        out_shape=[
            jax.ShapeDtypeStruct((B, T, n), jnp.float32),
            jax.ShapeDtypeStruct((B, T, n), jnp.float32),
            jax.ShapeDtypeStruct((B, T, n), jnp.float32),
            jax.ShapeDtypeStruct((B, T, R_C), jnp.float32),
            jax.ShapeDtypeStruct((B, T, R_F), jnp.float32),
        ],
        scratch_shapes=[pltpu.VMEM((4, n, c), jnp.float32)],
    )(x, oh)


def _normalize(h):
    mn = jnp.min(h)
    mx = jnp.max(h)
    return (h - mn) / (mx - mn + EPS)


def _kth_largest(score, k):
    hi0 = jnp.max(score) + 1.0
    lo0 = jnp.zeros((), jnp.float32)

    def body(_, carry):
        lo, hi = carry
        mid = 0.5 * (lo + hi)
        cnt = jnp.sum((score >= mid).astype(jnp.float32))
        ge = cnt >= float(k)
        return jnp.where(ge, mid, lo), jnp.where(ge, hi, mid)

    lo, _ = jax.lax.fori_loop(0, 50, body, (lo0, hi0))
    return lo


def _mask_kernel(ent1_ref, ent2_ref, ent4_ref, hc_ref, hf_ref,
                 oh_ref,
                 ghead_ref, gch_ref, gblock_ref, m2_ref, m4_ref,
                 mask_ref, headw_ref, chw_ref, blockw_ref, st_ref, last_ref):
    m2 = m2_ref[...]
    m4 = m4_ref[...]
    oh_c = oh_ref[:R_C, :]
    oh_f = oh_ref[R_C:, :]

    total = jnp.zeros((), jnp.float32)
    for b in range(B):
        e1 = ent1_ref[b]
        e2 = ent2_ref[b][1:T, :]
        e4 = ent4_ref[b][3:T, :]
        i2 = jnp.dot(m2, e2, preferred_element_type=jnp.float32, precision=jax.lax.Precision.HIGHEST)
        i4 = jnp.dot(m4, e4, preferred_element_type=jnp.float32, precision=jax.lax.Precision.HIGHEST)
        ht = (e1 + i2 + i4) * (1.0 / 3.0)
        ht_n = _normalize(ht)
        hc_n = _normalize(hc_ref[b])
        hf_n = _normalize(hf_ref[b])
        hc_tok = jnp.dot(hc_n, oh_c, preferred_element_type=jnp.float32, precision=jax.lax.Precision.HIGHEST)
        hf_tok = jnp.dot(hf_n, oh_f, preferred_element_type=jnp.float32, precision=jax.lax.Precision.HIGHEST)
        score = ALPHA * ht_n + BETA * hc_tok + GAMMA * hf_tok
        kth = _kth_largest(score, K_TOP)
        mask = jax.nn.sigmoid((score - kth) * (1.0 / TOK_TEMP))
        mask_ref[b] = mask
        total = total + jnp.sum(mask)

    sparsity_token = 1.0 - total / float(B * T * N_TOK)
    head_w = jax.nn.sigmoid(ghead_ref[...])
    ch_w = jax.nn.sigmoid(gch_ref[...])
    block_w = jax.nn.sigmoid(gblock_ref[...])
    headw_ref[...] = head_w
    chw_ref[...] = ch_w
    blockw_ref[...] = block_w
    l_ast = (sparsity_token + (1.0 - jnp.mean(head_w))
             + (1.0 - jnp.mean(ch_w)) + (1.0 - jnp.mean(block_w)))
    st_ref[...] = jnp.reshape(sparsity_token, (1, 1))
    last_ref[...] = jnp.reshape(l_ast, (1, 1))


def _mask_pass(ent1, ent2, ent4, hc, hf, oh,
               g_head, g_ch, g_block2d):
    return pl.pallas_call(
        _mask_kernel,
        out_shape=[
            jax.ShapeDtypeStruct((B, T, N_TOK), jnp.float32),
            jax.ShapeDtypeStruct((DEPTH, NUM_HEADS), jnp.float32),
            jax.ShapeDtypeStruct((DEPTH, HIDDEN_DIM), jnp.float32),
            jax.ShapeDtypeStruct((1, DEPTH), jnp.float32),
            jax.ShapeDtypeStruct((1, 1), jnp.float32),
            jax.ShapeDtypeStruct((1, 1), jnp.float32),
        ],
    )(ent1, ent2, ent4, hc, hf, oh, g_head, g_ch, g_block2d,
      jnp.asarray(M2_NP), jnp.asarray(M4_NP))


def kernel(token_feat, centers_coarse, centers_fine, g_head, g_ch, g_block,
           patch_coords):
    oh = jnp.concatenate([
        _region_one_hot(patch_coords, centers_coarse),
        _region_one_hot(patch_coords, centers_fine),
    ], axis=0)
    ent1, ent2, ent4, hc, hf = _entropy_pass(token_feat, oh)
    mask, head_w, ch_w, block_w2, st, last = _mask_pass(
        ent1, ent2, ent4, hc, hf, oh,
        g_head, g_ch, g_block.reshape(1, DEPTH))
    return (mask, head_w, ch_w, block_w2.reshape(DEPTH),
            st.reshape(()), last.reshape(()))
